# Initial kernel scaffold; baseline (speedup 1.0000x reference)
import jax
import jax.numpy as jnp
from jax.experimental import pallas as pl


def kernel(x, W_enc, W_dec, b_enc, b_dec):
    raise NotImplementedError("write your pallas kernel here")


# speedup vs baseline: 2.8866x; 2.8866x over previous
import jax
import jax.numpy as jnp
from jax import lax
from jax.experimental import pallas as pl
from jax.experimental.pallas import tpu as pltpu

B, S, D, H, K = 256, 3, 768, 16384, 64
SD = S * D
BH = 512
NH = H // BH
BISECT_ITERS = 44


def _body(x_ref, we_ref, wd_ref, be_ref, bd_ref, recon_ref, loss_ref,
          lat_ref, thr_ref):
    i = pl.program_id(0)

    @pl.when(i < NH)
    def _encode():
        blk = jnp.dot(x_ref[...], we_ref[...],
                      preferred_element_type=jnp.float32)
        blk = blk + be_ref[0, pl.ds(i * BH, BH)][None, :]
        lat_ref[:, pl.ds(i * BH, BH)] = blk

    @pl.when(i == NH - 1)
    def _threshold():
        lat = lat_ref[...]
        lo = jnp.min(lat, axis=1, keepdims=True) - 1.0
        hi = jnp.max(lat, axis=1, keepdims=True) + 1.0

        def it(_, carry):
            lo, hi = carry
            t = 0.5 * (lo + hi)
            c = jnp.sum((lat >= t).astype(jnp.float32), axis=1,
                        keepdims=True)
            pred = c >= K
            return (jnp.where(pred, t, lo), jnp.where(pred, hi, t))

        lo, hi = lax.fori_loop(0, BISECT_ITERS, it, (lo, hi))
        thr_ref[...] = lo

    @pl.when(i >= NH)
    def _decode():
        j = i - NH
        latb = lat_ref[:, pl.ds(j * BH, BH)]
        mid = jnp.where(latb >= thr_ref[...], jnp.maximum(latb, 0.0), 0.0)
        contrib = jnp.dot(mid, wd_ref[...],
                          preferred_element_type=jnp.float32)

        @pl.when(j == 0)
        def _init():
            recon_ref[...] = contrib

        @pl.when(j > 0)
        def _acc():
            recon_ref[...] = recon_ref[...] + contrib

    @pl.when(i == 2 * NH - 1)
    def _finish():
        rec = recon_ref[...] + bd_ref[...]
        recon_ref[...] = rec
        diff = x_ref[...] - rec
        sq = diff * diff
        s0 = jnp.sum(sq[:, :D])
        s1 = jnp.sum(sq[:, D:2 * D])
        s2 = jnp.sum(sq[:, 2 * D:])
        denom = float(B * D)
        loss_ref[0] = (s0 + s1 + s2) / (3.0 * denom)
        loss_ref[1] = s0 / denom
        loss_ref[2] = s1 / denom
        loss_ref[3] = s2 / denom


def kernel(x, W_enc, W_dec, b_enc, b_dec):
    xf = x.reshape(B, SD)
    wef = W_enc.reshape(SD, H)
    wdf = W_dec.reshape(H, SD)
    bef = b_enc.reshape(1, H)
    bdf = b_dec.reshape(1, SD)
    recon, losses = pl.pallas_call(
        _body,
        grid=(2 * NH,),
        in_specs=[
            pl.BlockSpec((B, SD), lambda i: (0, 0)),
            pl.BlockSpec((SD, BH), lambda i: (0, jnp.minimum(i, NH - 1))),
            pl.BlockSpec((BH, SD), lambda i: (jnp.maximum(i - NH, 0), 0)),
            pl.BlockSpec((1, H), lambda i: (0, 0)),
            pl.BlockSpec((1, SD), lambda i: (0, 0)),
        ],
        out_specs=[
            pl.BlockSpec((B, SD), lambda i: (0, 0)),
            pl.BlockSpec(memory_space=pltpu.SMEM),
        ],
        out_shape=[
            jax.ShapeDtypeStruct((B, SD), jnp.float32),
            jax.ShapeDtypeStruct((4,), jnp.float32),
        ],
        scratch_shapes=[
            pltpu.VMEM((B, H), jnp.float32),
            pltpu.VMEM((B, 1), jnp.float32),
        ],
        compiler_params=pltpu.CompilerParams(
            dimension_semantics=("arbitrary",),
            vmem_limit_bytes=58 * 1024 * 1024,
        ),
    )(xf, wef, wdf, bef, bdf)
    aux = jnp.zeros((), jnp.float32)
    return (losses[0], aux, losses[1], losses[2], losses[3],
            recon.reshape(B, S, D))


# speedup vs baseline: 2.9896x; 1.0357x over previous
import jax
import jax.numpy as jnp
from jax import lax
from jax.experimental import pallas as pl
from jax.experimental.pallas import tpu as pltpu

B, S, D, H, K = 256, 3, 768, 16384, 64
SD = S * D
BH = 512
NH = H // BH
BISECT_ITERS = 36


def _body(x_ref, we_ref, wd_ref, be_ref, bd_ref, recon_ref, loss_ref,
          lat_ref, thr_ref):
    i = pl.program_id(0)

    @pl.when(i < NH)
    def _encode():
        blk = jnp.dot(x_ref[...], we_ref[...],
                      preferred_element_type=jnp.float32)
        blk = blk + be_ref[0, pl.ds(i * BH, BH)][None, :]
        lat_ref[:, pl.ds(i * BH, BH)] = blk

    @pl.when(i == NH - 1)
    def _threshold():
        lat = lat_ref[...]
        lo = jnp.min(lat, axis=1, keepdims=True) - 1.0
        hi = jnp.max(lat, axis=1, keepdims=True) + 1.0

        def it(_, carry):
            lo, hi = carry
            t = 0.5 * (lo + hi)
            c = jnp.sum((lat >= t).astype(jnp.float32), axis=1,
                        keepdims=True)
            pred = c >= K
            return (jnp.where(pred, t, lo), jnp.where(pred, hi, t))

        lo, hi = lax.fori_loop(0, BISECT_ITERS, it, (lo, hi))
        thr_ref[...] = lo

    @pl.when(i >= NH)
    def _decode():
        j = i - NH
        latb = lat_ref[:, pl.ds(j * BH, BH)]
        mid = jnp.where(latb >= thr_ref[...], jnp.maximum(latb, 0.0), 0.0)
        contrib = jnp.dot(mid.astype(jnp.bfloat16),
                          wd_ref[...].astype(jnp.bfloat16),
                          preferred_element_type=jnp.float32)

        @pl.when(j == 0)
        def _init():
            recon_ref[...] = contrib

        @pl.when(j > 0)
        def _acc():
            recon_ref[...] = recon_ref[...] + contrib

    @pl.when(i == 2 * NH - 1)
    def _finish():
        rec = recon_ref[...] + bd_ref[...]
        recon_ref[...] = rec
        diff = x_ref[...] - rec
        sq = diff * diff
        s0 = jnp.sum(sq[:, :D])
        s1 = jnp.sum(sq[:, D:2 * D])
        s2 = jnp.sum(sq[:, 2 * D:])
        denom = float(B * D)
        loss_ref[0] = (s0 + s1 + s2) / (3.0 * denom)
        loss_ref[1] = s0 / denom
        loss_ref[2] = s1 / denom
        loss_ref[3] = s2 / denom


def kernel(x, W_enc, W_dec, b_enc, b_dec):
    xf = x.reshape(B, SD)
    wef = W_enc.reshape(SD, H)
    wdf = W_dec.reshape(H, SD)
    bef = b_enc.reshape(1, H)
    bdf = b_dec.reshape(1, SD)
    recon, losses = pl.pallas_call(
        _body,
        grid=(2 * NH,),
        in_specs=[
            pl.BlockSpec((B, SD), lambda i: (0, 0)),
            pl.BlockSpec((SD, BH), lambda i: (0, jnp.minimum(i, NH - 1))),
            pl.BlockSpec((BH, SD), lambda i: (jnp.maximum(i - NH, 0), 0)),
            pl.BlockSpec((1, H), lambda i: (0, 0)),
            pl.BlockSpec((1, SD), lambda i: (0, 0)),
        ],
        out_specs=[
            pl.BlockSpec((B, SD), lambda i: (0, 0)),
            pl.BlockSpec(memory_space=pltpu.SMEM),
        ],
        out_shape=[
            jax.ShapeDtypeStruct((B, SD), jnp.float32),
            jax.ShapeDtypeStruct((4,), jnp.float32),
        ],
        scratch_shapes=[
            pltpu.VMEM((B, H), jnp.float32),
            pltpu.VMEM((B, 1), jnp.float32),
        ],
        compiler_params=pltpu.CompilerParams(
            dimension_semantics=("arbitrary",),
            vmem_limit_bytes=58 * 1024 * 1024,
        ),
    )(xf, wef, wdf, bef, bdf)
    aux = jnp.zeros((), jnp.float32)
    return (losses[0], aux, losses[1], losses[2], losses[3],
            recon.reshape(B, S, D))


# speedup vs baseline: 3.0396x; 1.0167x over previous
import jax
import jax.numpy as jnp
from jax import lax
from jax.experimental import pallas as pl
from jax.experimental.pallas import tpu as pltpu

B, S, D, H, K = 256, 3, 768, 16384, 64
SD = S * D
BHE = 1024
BHD = 512
NHE = H // BHE
NHD = H // BHD
BISECT_ITERS = 36


def _body(x_ref, we_ref, wd_ref, be_ref, bd_ref, recon_ref, loss_ref,
          lat_ref, thr_ref):
    i = pl.program_id(0)

    @pl.when(i < NHE)
    def _encode():
        blk = jnp.dot(x_ref[...], we_ref[...],
                      preferred_element_type=jnp.float32)
        blk = blk + be_ref[0, pl.ds(i * BHE, BHE)][None, :]
        lat_ref[:, pl.ds(i * BHE, BHE)] = blk

    @pl.when(i == NHE - 1)
    def _threshold():
        lo = jnp.min(lat_ref[...], axis=1, keepdims=True) - 1.0
        hi = jnp.max(lat_ref[...], axis=1, keepdims=True) + 1.0

        def it(_, carry):
            lo, hi = carry
            t = 0.5 * (lo + hi)
            c = jnp.sum((lat_ref[...] >= t).astype(jnp.float32), axis=1,
                        keepdims=True)
            pred = c >= K
            return (jnp.where(pred, t, lo), jnp.where(pred, hi, t))

        lo, hi = lax.fori_loop(0, BISECT_ITERS, it, (lo, hi))
        thr_ref[...] = lo

    @pl.when(i >= NHE)
    def _decode():
        j = i - NHE
        latb = lat_ref[:, pl.ds(j * BHD, BHD)]
        mid = jnp.where(latb >= thr_ref[...], jnp.maximum(latb, 0.0), 0.0)
        contrib = jnp.dot(mid, wd_ref[...],
                          preferred_element_type=jnp.float32)

        @pl.when(j == 0)
        def _init():
            recon_ref[...] = contrib

        @pl.when(j > 0)
        def _acc():
            recon_ref[...] = recon_ref[...] + contrib

    @pl.when(i == NHE + NHD - 1)
    def _finish():
        rec = recon_ref[...] + bd_ref[...]
        recon_ref[...] = rec
        diff = x_ref[...] - rec
        sq = diff * diff
        s0 = jnp.sum(sq[:, :D])
        s1 = jnp.sum(sq[:, D:2 * D])
        s2 = jnp.sum(sq[:, 2 * D:])
        denom = float(B * D)
        loss_ref[0] = (s0 + s1 + s2) / (3.0 * denom)
        loss_ref[1] = s0 / denom
        loss_ref[2] = s1 / denom
        loss_ref[3] = s2 / denom


def kernel(x, W_enc, W_dec, b_enc, b_dec):
    xf = x.reshape(B, SD)
    wef = W_enc.reshape(SD, H)
    wdf = W_dec.reshape(H, SD)
    bef = b_enc.reshape(1, H)
    bdf = b_dec.reshape(1, SD)
    recon, losses = pl.pallas_call(
        _body,
        grid=(NHE + NHD,),
        in_specs=[
            pl.BlockSpec((B, SD), lambda i: (0, 0)),
            pl.BlockSpec((SD, BHE), lambda i: (0, jnp.minimum(i, NHE - 1))),
            pl.BlockSpec((BHD, SD), lambda i: (jnp.maximum(i - NHE, 0), 0)),
            pl.BlockSpec((1, H), lambda i: (0, 0)),
            pl.BlockSpec((1, SD), lambda i: (0, 0)),
        ],
        out_specs=[
            pl.BlockSpec((B, SD), lambda i: (0, 0)),
            pl.BlockSpec(memory_space=pltpu.SMEM),
        ],
        out_shape=[
            jax.ShapeDtypeStruct((B, SD), jnp.float32),
            jax.ShapeDtypeStruct((4,), jnp.float32),
        ],
        scratch_shapes=[
            pltpu.VMEM((B, H), jnp.float32),
            pltpu.VMEM((B, 1), jnp.float32),
        ],
        compiler_params=pltpu.CompilerParams(
            dimension_semantics=("arbitrary",),
            vmem_limit_bytes=60000 * 1024,
        ),
    )(xf, wef, wdf, bef, bdf)
    aux = jnp.zeros((), jnp.float32)
    return (losses[0], aux, losses[1], losses[2], losses[3],
            recon.reshape(B, S, D))
